# baseline (speedup 1.0000x reference)
import functools

import jax
import jax.numpy as jnp
from jax import lax
from jax.experimental import pallas as pl
from jax.experimental.pallas import tpu as pltpu
from jax.experimental.pallas import tpu_sc as plsc

_N_EMBED = 64
_BATCH = 16384
_N_SAMPLES = 5
_NC, _NS = 2, 16
_NW = _NC * _NS
_CHUNK = 128

_B_IN = _BATCH // _NW
_B_NZ = _BATCH * _N_SAMPLES // _NW
_C_IN = _B_IN // _CHUNK
_C_NZ = _B_NZ // _CHUNK


def _sg_body(in_table, out_table, idx_in, idx_out, idx_nz,
             o_in, o_out, o_nz, v_in, v_out, v_nz, rows, gsem):
    wid = lax.axis_index("s") * _NC + lax.axis_index("c")

    pltpu.sync_copy(idx_in.at[wid], v_in)
    pltpu.sync_copy(idx_out.at[wid], v_out)
    pltpu.sync_copy(idx_nz.at[wid], v_nz)

    def run(table, vidx, nchunks, out, base):
        def step(j, carry):
            pltpu.async_copy(table.at[vidx.at[j]], rows, gsem).wait()
            pltpu.sync_copy(rows, out.at[pl.ds(base + j * _CHUNK, _CHUNK)])
            return carry
        lax.fori_loop(0, nchunks, step, 0)

    run(in_table, v_in, _C_IN, o_in, wid * _B_IN)
    run(out_table, v_out, _C_IN, o_out, wid * _B_IN)
    run(out_table, v_nz, _C_NZ, o_nz, wid * _B_NZ)


@jax.jit
def _sg_gather(iw, ow, nz, in_table, out_table):
    mesh = plsc.VectorSubcoreMesh(core_axis_name="c", subcore_axis_name="s")
    f = functools.partial(
        pl.kernel,
        mesh=mesh,
        compiler_params=pltpu.CompilerParams(use_tc_tiling_on_sc=False),
        out_type=(
            jax.ShapeDtypeStruct((_BATCH, _N_EMBED), jnp.float32),
            jax.ShapeDtypeStruct((_BATCH, _N_EMBED), jnp.float32),
            jax.ShapeDtypeStruct((_BATCH * _N_SAMPLES, _N_EMBED), jnp.float32),
        ),
        scratch_types=[
            pltpu.VMEM((_C_IN, _CHUNK), jnp.int32),
            pltpu.VMEM((_C_IN, _CHUNK), jnp.int32),
            pltpu.VMEM((_C_NZ, _CHUNK), jnp.int32),
            pltpu.VMEM((_CHUNK, _N_EMBED), jnp.float32),
            pltpu.SemaphoreType.DMA,
        ],
    )(_sg_body)
    return f(in_table, out_table, iw, ow, nz)


def kernel(input_words, output_words, noise_words, in_table, out_table):
    iw = input_words.astype(jnp.int32).reshape(_NW, _C_IN, _CHUNK)
    ow = output_words.astype(jnp.int32).reshape(_NW, _C_IN, _CHUNK)
    nz = noise_words.astype(jnp.int32).reshape(_NW, _C_NZ, _CHUNK)
    iv, ov, nv = _sg_gather(iw, ow, nz, in_table, out_table)
    return iv, ov, nv.reshape(_BATCH, _N_SAMPLES, _N_EMBED)


# speedup vs baseline: 1.0142x; 1.0142x over previous
import functools

import jax
import jax.numpy as jnp
from jax import lax
from jax.experimental import pallas as pl
from jax.experimental.pallas import tpu as pltpu
from jax.experimental.pallas import tpu_sc as plsc

_N_EMBED = 64
_BATCH = 16384
_N_SAMPLES = 5
_NC, _NS = 2, 16
_NW = _NC * _NS
_CHUNK = 128
_SUPER = 512
_NBUF = 3

_B_IN = _BATCH // _NW
_B_NZ = _BATCH * _N_SAMPLES // _NW
_C_IN = _B_IN // _CHUNK
_C_NZ = _B_NZ // _CHUNK
_S_IN = _B_IN // _SUPER
_S_NZ = _B_NZ // _SUPER
_NTASK = 2 * _S_IN + _S_NZ


def _sg_body(in_table, out_table, idx_in, idx_out, idx_nz,
             o_in, o_out, o_nz, v_in, v_out, v_nz,
             b0, b1, b2, g0, g1, g2, s0, s1, s2):
    wid = lax.axis_index("s") * _NC + lax.axis_index("c")
    bufs = (b0, b1, b2)
    gsems = (g0, g1, g2)
    ssems = (s0, s1, s2)

    pltpu.sync_copy(idx_in.at[wid], v_in)
    pltpu.sync_copy(idx_out.at[wid], v_out)
    pltpu.sync_copy(idx_nz.at[wid], v_nz)

    tasks = []
    for s in range(_S_IN):
        tasks.append((in_table, v_in, 4 * s, o_in, wid * _B_IN + s * _SUPER))
    for s in range(_S_IN):
        tasks.append((out_table, v_out, 4 * s, o_out, wid * _B_IN + s * _SUPER))
    for s in range(_S_NZ):
        tasks.append((out_table, v_nz, 4 * s, o_nz, wid * _B_NZ + s * _SUPER))

    def fire_gathers(t):
        table, vidx, crow, _, _ = tasks[t]
        b = t % _NBUF
        return [
            pltpu.async_copy(
                table.at[vidx.at[crow + k]],
                bufs[b].at[pl.ds(k * _CHUNK, _CHUNK)],
                gsems[b],
            )
            for k in range(4)
        ]

    def fire_store(t):
        _, _, _, out, orow = tasks[t]
        b = t % _NBUF
        return pltpu.async_copy(bufs[b], out.at[pl.ds(orow, _SUPER)], ssems[b])

    gd = [None] * _NTASK
    sd = [None] * _NTASK
    gd[0] = fire_gathers(0)
    gd[1] = fire_gathers(1)
    for t in range(_NTASK):
        for d in gd[t]:
            d.wait()
        sd[t] = fire_store(t)
        u = t + 2
        if u < _NTASK:
            if u >= _NBUF:
                sd[u - _NBUF].wait()
            gd[u] = fire_gathers(u)
    for t in range(_NTASK - _NBUF, _NTASK):
        sd[t].wait()


@jax.jit
def _sg_gather(iw, ow, nz, in_table, out_table):
    mesh = plsc.VectorSubcoreMesh(core_axis_name="c", subcore_axis_name="s")
    f = functools.partial(
        pl.kernel,
        mesh=mesh,
        compiler_params=pltpu.CompilerParams(use_tc_tiling_on_sc=False),
        out_type=(
            jax.ShapeDtypeStruct((_BATCH, _N_EMBED), jnp.float32),
            jax.ShapeDtypeStruct((_BATCH, _N_EMBED), jnp.float32),
            jax.ShapeDtypeStruct((_BATCH * _N_SAMPLES, _N_EMBED), jnp.float32),
        ),
        scratch_types=[
            pltpu.VMEM((_C_IN, _CHUNK), jnp.int32),
            pltpu.VMEM((_C_IN, _CHUNK), jnp.int32),
            pltpu.VMEM((_C_NZ, _CHUNK), jnp.int32),
            pltpu.VMEM((_SUPER, _N_EMBED), jnp.float32),
            pltpu.VMEM((_SUPER, _N_EMBED), jnp.float32),
            pltpu.VMEM((_SUPER, _N_EMBED), jnp.float32),
            pltpu.SemaphoreType.DMA,
            pltpu.SemaphoreType.DMA,
            pltpu.SemaphoreType.DMA,
            pltpu.SemaphoreType.DMA,
            pltpu.SemaphoreType.DMA,
            pltpu.SemaphoreType.DMA,
        ],
    )(_sg_body)
    return f(in_table, out_table, iw, ow, nz)


def kernel(input_words, output_words, noise_words, in_table, out_table):
    iw = input_words.astype(jnp.int32).reshape(_NW, _C_IN, _CHUNK)
    ow = output_words.astype(jnp.int32).reshape(_NW, _C_IN, _CHUNK)
    nz = noise_words.astype(jnp.int32).reshape(_NW, _C_NZ, _CHUNK)
    iv, ov, nv = _sg_gather(iw, ow, nz, in_table, out_table)
    return iv, ov, nv.reshape(_BATCH, _N_SAMPLES, _N_EMBED)
